# baseline (speedup 1.0000x reference)
import functools

import jax
import jax.numpy as jnp
from jax import lax
from jax.experimental import pallas as pl
from jax.experimental.pallas import tpu as pltpu, tpu_sc as plsc

_BS = 32
_N = 200
_D = 128
_F = 512
_K = 2048
_B = _BS * _N
_NW = 32
_BPW = _B // _NW


def _tc_body(x_ref, nk_ref, ak_ref, nkab_ref, akab_ref, colhl_ref,
             wc_ref, bc_ref, wp_ref, bp_ref,
             wpm1_ref, wpm2_ref, bpm_ref,
             rn_ref, p_ref, up_ref, ia_ref, ib_ref):
    x = x_ref[0]
    z0 = jnp.dot(x, wc_ref[0], preferred_element_type=jnp.float32)
    z1 = jnp.dot(x, wc_ref[1], preferred_element_type=jnp.float32)
    z2 = jnp.dot(x, wc_ref[2], preferred_element_type=jnp.float32)
    zrow = jnp.zeros((1, _D), jnp.float32)
    y = z1 + jnp.concatenate([zrow, z0[:-1]], axis=0) \
           + jnp.concatenate([z2[1:], zrow], axis=0)
    q = jnp.maximum(y + bc_ref[...], 0.0)
    rn_ref[0] = q

    nk = nk_ref[...]
    ak = ak_ref[...]
    dn = (((1,), (1,)), ((), ()))
    sn = lax.dot_general(q, nk, dn, preferred_element_type=jnp.float32)
    sa = lax.dot_general(q, ak, dn, preferred_element_type=jnp.float32)

    mn = jnp.max(sn, axis=1, keepdims=True)
    ma = jnp.max(sa, axis=1, keepdims=True)
    maskn = jnp.where(sn >= mn, 1.0, 0.0).astype(jnp.bfloat16)
    maska = jnp.where(sa >= ma, 1.0, 0.0).astype(jnp.bfloat16)
    colhl = colhl_ref[...]
    rn_idx = lax.dot_general(maskn, colhl, dn, preferred_element_type=jnp.float32)
    ra_idx = lax.dot_general(maska, colhl, dn, preferred_element_type=jnp.float32)
    iaf = 128.0 * rn_idx[:, 0:1] + rn_idx[:, 1:2]
    ibf = 128.0 * ra_idx[:, 0:1] + ra_idx[:, 1:2]
    ia = jnp.clip(iaf, 0.0, float(_K - 1)).astype(jnp.int32)
    ib = jnp.clip(ibf, 0.0, float(_K - 1)).astype(jnp.int32)

    m = jnp.maximum(mn, ma)
    pnb = jnp.exp(sn - m).astype(jnp.bfloat16)
    pab = jnp.exp(sa - m).astype(jnp.bfloat16)
    cmn = jnp.dot(pnb, nkab_ref[...], preferred_element_type=jnp.float32)
    cma = jnp.dot(pab, akab_ref[...], preferred_element_type=jnp.float32)
    l = cmn[:, _D:] + cma[:, _D:]
    cm = (cmn[:, :_D] + cma[:, :_D]) / l

    sp = jnp.sum(q * wp_ref[...], axis=1, keepdims=True) + bp_ref[...]
    p = 1.0 / (1.0 + jnp.exp(-sp))
    su = (jnp.sum(q * wpm1_ref[...], axis=1, keepdims=True)
          + jnp.sum(cm * wpm2_ref[...], axis=1, keepdims=True) + bpm_ref[...])
    up = 1.0 / (1.0 + jnp.exp(-su))

    p_ref[0] = p
    up_ref[0] = up
    ia_ref[0] = ia
    ib_ref[0] = ib


def _tc_call(x, nk, ak, nkab, akab, colhl, wc, bc, wp, bp, wpm1, wpm2, bpm):
    const2 = lambda b: (0, 0)
    const3 = lambda b: (0, 0, 0)
    return pl.pallas_call(
        _tc_body,
        grid=(_BS,),
        in_specs=[
            pl.BlockSpec((1, _N, _F), lambda b: (b, 0, 0)),
            pl.BlockSpec((_K, _D), const2),
            pl.BlockSpec((_K, _D), const2),
            pl.BlockSpec((_K, _D + 1), const2),
            pl.BlockSpec((_K, _D + 1), const2),
            pl.BlockSpec((2, _K), const2),
            pl.BlockSpec((3, _F, _D), const3),
            pl.BlockSpec((1, _D), const2),
            pl.BlockSpec((1, _D), const2),
            pl.BlockSpec((1, 1), const2),
            pl.BlockSpec((1, _D), const2),
            pl.BlockSpec((1, _D), const2),
            pl.BlockSpec((1, 1), const2),
        ],
        out_specs=[
            pl.BlockSpec((1, _N, _D), lambda b: (b, 0, 0)),
            pl.BlockSpec((1, _N, 1), lambda b: (b, 0, 0)),
            pl.BlockSpec((1, _N, 1), lambda b: (b, 0, 0)),
            pl.BlockSpec((1, _N, 1), lambda b: (b, 0, 0)),
            pl.BlockSpec((1, _N, 1), lambda b: (b, 0, 0)),
        ],
        out_shape=[
            jax.ShapeDtypeStruct((_BS, _N, _D), jnp.float32),
            jax.ShapeDtypeStruct((_BS, _N, 1), jnp.float32),
            jax.ShapeDtypeStruct((_BS, _N, 1), jnp.float32),
            jax.ShapeDtypeStruct((_BS, _N, 1), jnp.int32),
            jax.ShapeDtypeStruct((_BS, _N, 1), jnp.int32),
        ],
        compiler_params=pltpu.CompilerParams(
            dimension_semantics=("arbitrary",),
        ),
    )(x, nk, ak, nkab, akab, colhl, wc, bc, wp, bp, wpm1, wpm2, bpm)


def _sc_loss_body(q_hbm, nk_hbm, ak_hbm, ian_hbm, iab_hbm, outn_hbm, outa_hbm,
                  idxn_v, idxa_v, q_v, kn_v, ka_v, on_v, oa_v, sem):
    wid = lax.axis_index("s") * 2 + lax.axis_index("c")
    base = wid * _BPW
    pltpu.sync_copy(ian_hbm.at[pl.ds(base, _BPW)], idxn_v)
    pltpu.sync_copy(iab_hbm.at[pl.ds(base, _BPW)], idxa_v)
    descs = []
    for off, sz in ((0, 104), (104, 96)):
        descs.append(pltpu.async_copy(
            nk_hbm.at[idxn_v.at[pl.ds(off, sz)]], kn_v.at[pl.ds(off, sz)], sem))
        descs.append(pltpu.async_copy(
            ak_hbm.at[idxa_v.at[pl.ds(off, sz)]], ka_v.at[pl.ds(off, sz)], sem))
    pltpu.sync_copy(q_hbm.at[pl.ds(base, _BPW)], q_v)
    for d in descs:
        d.wait()

    lane = lax.broadcasted_iota(jnp.int32, (16,), 0)
    mask0 = lane == 0

    def lane_total(v):
        s = v
        for sh in (8, 4, 2, 1):
            s = s + s.at[(lane + sh) & 15].get(mode="promise_in_bounds")
        return s

    def mse_row(i):
        an = [jnp.zeros((16,), jnp.float32) for _ in range(4)]
        aa = [jnp.zeros((16,), jnp.float32) for _ in range(4)]
        for c in range(_D // 16):
            sl = pl.ds(c * 16, 16)
            qv = q_v[i, sl]
            dnv = qv - kn_v[i, sl]
            dav = qv - ka_v[i, sl]
            an[c % 4] = an[c % 4] + dnv * dnv
            aa[c % 4] = aa[c % 4] + dav * dav
        tn = lane_total((an[0] + an[1]) + (an[2] + an[3])) * (1.0 / _D)
        ta = lane_total((aa[0] + aa[1]) + (aa[2] + aa[3])) * (1.0 / _D)
        iv = jnp.full((16,), i, jnp.int32)
        plsc.store_scatter(on_v, [iv], tn, mask=mask0)
        plsc.store_scatter(oa_v, [iv], ta, mask=mask0)

    def row(i, carry):
        mse_row(2 * i)
        mse_row(2 * i + 1)
        return carry

    lax.fori_loop(0, _BPW // 2, row, 0)
    pltpu.sync_copy(on_v, outn_hbm.at[pl.ds(base, _BPW)])
    pltpu.sync_copy(oa_v, outa_hbm.at[pl.ds(base, _BPW)])


@functools.cache
def _sc_loss_kernel():
    mesh = plsc.VectorSubcoreMesh(core_axis_name="c", subcore_axis_name="s")
    return pl.kernel(
        _sc_loss_body,
        out_type=(jax.ShapeDtypeStruct((_B,), jnp.float32),
                  jax.ShapeDtypeStruct((_B,), jnp.float32)),
        mesh=mesh,
        scratch_types=[
            pltpu.VMEM((_BPW,), jnp.int32),
            pltpu.VMEM((_BPW,), jnp.int32),
            pltpu.VMEM((_BPW, _D), jnp.float32),
            pltpu.VMEM((_BPW, _D), jnp.float32),
            pltpu.VMEM((_BPW, _D), jnp.float32),
            pltpu.VMEM((_BPW,), jnp.float32),
            pltpu.VMEM((_BPW,), jnp.float32),
            pltpu.SemaphoreType.DMA,
        ],
        compiler_params=pltpu.CompilerParams(needs_layout_passes=False),
    )


def kernel(ref_nor, ref_abn, nor_keys, abn_keys, W_conv, b_conv, W_p, b_p,
           W_pm, b_pm, epoch, isTrain):
    wc = jnp.transpose(W_conv, (2, 1, 0))
    ones_col = jnp.ones((_K, 1), jnp.bfloat16)
    nkab = jnp.concatenate([nor_keys.astype(jnp.bfloat16), ones_col], axis=1)
    akab = jnp.concatenate([abn_keys.astype(jnp.bfloat16), ones_col], axis=1)
    coli = jnp.arange(_K, dtype=jnp.int32)
    colhl = jnp.stack([(coli >> 7).astype(jnp.bfloat16),
                       (coli & 127).astype(jnp.bfloat16)], axis=0)
    bc = b_conv.reshape(1, _D)
    wp = W_p.reshape(1, _D)
    bp = b_p.reshape(1, 1)
    wpm1 = W_pm[:, :_D].reshape(1, _D)
    wpm2 = W_pm[:, _D:].reshape(1, _D)
    bpm = b_pm.reshape(1, 1)

    rn, p, up, ia, ib = _tc_call(ref_nor, nor_keys, abn_keys, nkab, akab,
                                 colhl, wc, bc, wp, bp, wpm1, wpm2, bpm)

    q = rn.reshape(_B, _D)
    ln, la = _sc_loss_kernel()(q, nor_keys, abn_keys,
                               ia.reshape(_B), ib.reshape(_B))

    return (p.reshape(_BS, _N), up.reshape(_BS, _N), ln, la, rn)


# speedup vs baseline: 1.0232x; 1.0232x over previous
import functools

import jax
import jax.numpy as jnp
from jax import lax
from jax.experimental import pallas as pl
from jax.experimental.pallas import tpu as pltpu, tpu_sc as plsc

_BS = 32
_N = 200
_D = 128
_F = 512
_K = 2048
_B = _BS * _N
_NW = 32
_BPW = _B // _NW


def _tc_body(x_ref, nk_ref, ak_ref, nkab_ref, akab_ref, colhl_ref,
             wc_ref, bc_ref, wp_ref, bp_ref,
             wpm1_ref, wpm2_ref, bpm_ref,
             rn_ref, p_ref, up_ref, ia_ref, ib_ref, mn_ref, ma_ref, qn_ref):
    x = x_ref[0]
    z0 = jnp.dot(x, wc_ref[0], preferred_element_type=jnp.float32)
    z1 = jnp.dot(x, wc_ref[1], preferred_element_type=jnp.float32)
    z2 = jnp.dot(x, wc_ref[2], preferred_element_type=jnp.float32)
    zrow = jnp.zeros((1, _D), jnp.float32)
    y = z1 + jnp.concatenate([zrow, z0[:-1]], axis=0) \
           + jnp.concatenate([z2[1:], zrow], axis=0)
    q = jnp.maximum(y + bc_ref[...], 0.0)
    rn_ref[0] = q

    nk = nk_ref[...]
    ak = ak_ref[...]
    dn = (((1,), (1,)), ((), ()))
    sn = lax.dot_general(q, nk, dn, preferred_element_type=jnp.float32)
    sa = lax.dot_general(q, ak, dn, preferred_element_type=jnp.float32)

    mn = jnp.max(sn, axis=1, keepdims=True)
    ma = jnp.max(sa, axis=1, keepdims=True)
    maskn = jnp.where(sn >= mn, 1.0, 0.0).astype(jnp.bfloat16)
    maska = jnp.where(sa >= ma, 1.0, 0.0).astype(jnp.bfloat16)
    colhl = colhl_ref[...]
    rn_idx = lax.dot_general(maskn, colhl, dn, preferred_element_type=jnp.float32)
    ra_idx = lax.dot_general(maska, colhl, dn, preferred_element_type=jnp.float32)
    iaf = 128.0 * rn_idx[:, 0:1] + rn_idx[:, 1:2]
    ibf = 128.0 * ra_idx[:, 0:1] + ra_idx[:, 1:2]
    ia = jnp.clip(iaf, 0.0, float(_K - 1)).astype(jnp.int32)
    ib = jnp.clip(ibf, 0.0, float(_K - 1)).astype(jnp.int32)

    m = jnp.maximum(mn, ma)
    pnb = jnp.exp(sn - m).astype(jnp.bfloat16)
    pab = jnp.exp(sa - m).astype(jnp.bfloat16)
    cmn = jnp.dot(pnb, nkab_ref[...], preferred_element_type=jnp.float32)
    cma = jnp.dot(pab, akab_ref[...], preferred_element_type=jnp.float32)
    l = cmn[:, _D:] + cma[:, _D:]
    cm = (cmn[:, :_D] + cma[:, :_D]) / l

    sp = jnp.sum(q * wp_ref[...], axis=1, keepdims=True) + bp_ref[...]
    p = 1.0 / (1.0 + jnp.exp(-sp))
    su = (jnp.sum(q * wpm1_ref[...], axis=1, keepdims=True)
          + jnp.sum(cm * wpm2_ref[...], axis=1, keepdims=True) + bpm_ref[...])
    up = 1.0 / (1.0 + jnp.exp(-su))

    p_ref[0] = p
    up_ref[0] = up
    ia_ref[0] = ia
    ib_ref[0] = ib
    mn_ref[0] = mn
    ma_ref[0] = ma
    qn_ref[0] = jnp.sum(q * q, axis=1, keepdims=True)


def _tc_call(x, nk, ak, nkab, akab, colhl, wc, bc, wp, bp, wpm1, wpm2, bpm):
    const2 = lambda b: (0, 0)
    const3 = lambda b: (0, 0, 0)
    return pl.pallas_call(
        _tc_body,
        grid=(_BS,),
        in_specs=[
            pl.BlockSpec((1, _N, _F), lambda b: (b, 0, 0)),
            pl.BlockSpec((_K, _D), const2),
            pl.BlockSpec((_K, _D), const2),
            pl.BlockSpec((_K, _D + 1), const2),
            pl.BlockSpec((_K, _D + 1), const2),
            pl.BlockSpec((2, _K), const2),
            pl.BlockSpec((3, _F, _D), const3),
            pl.BlockSpec((1, _D), const2),
            pl.BlockSpec((1, _D), const2),
            pl.BlockSpec((1, 1), const2),
            pl.BlockSpec((1, _D), const2),
            pl.BlockSpec((1, _D), const2),
            pl.BlockSpec((1, 1), const2),
        ],
        out_specs=[
            pl.BlockSpec((1, _N, _D), lambda b: (b, 0, 0)),
            pl.BlockSpec((1, _N, 1), lambda b: (b, 0, 0)),
            pl.BlockSpec((1, _N, 1), lambda b: (b, 0, 0)),
            pl.BlockSpec((1, _N, 1), lambda b: (b, 0, 0)),
            pl.BlockSpec((1, _N, 1), lambda b: (b, 0, 0)),
            pl.BlockSpec((1, _N, 1), lambda b: (b, 0, 0)),
            pl.BlockSpec((1, _N, 1), lambda b: (b, 0, 0)),
            pl.BlockSpec((1, _N, 1), lambda b: (b, 0, 0)),
        ],
        out_shape=[
            jax.ShapeDtypeStruct((_BS, _N, _D), jnp.float32),
            jax.ShapeDtypeStruct((_BS, _N, 1), jnp.float32),
            jax.ShapeDtypeStruct((_BS, _N, 1), jnp.float32),
            jax.ShapeDtypeStruct((_BS, _N, 1), jnp.int32),
            jax.ShapeDtypeStruct((_BS, _N, 1), jnp.int32),
            jax.ShapeDtypeStruct((_BS, _N, 1), jnp.float32),
            jax.ShapeDtypeStruct((_BS, _N, 1), jnp.float32),
            jax.ShapeDtypeStruct((_BS, _N, 1), jnp.float32),
        ],
        compiler_params=pltpu.CompilerParams(
            dimension_semantics=("arbitrary",),
        ),
    )(x, nk, ak, nkab, akab, colhl, wc, bc, wp, bp, wpm1, wpm2, bpm)


_BPWP = 208


def _sc_loss_body(mn_hbm, ma_hbm, qn_hbm, ian_hbm, iab_hbm, nn_hbm, na_hbm,
                  outn_hbm, outa_hbm,
                  idxn_v, idxa_v, mn_v, ma_v, qn_v, nn_v, na_v, on_v, oa_v):
    wid = lax.axis_index("s") * 2 + lax.axis_index("c")
    base = wid * _BPW
    head = pl.ds(0, _BPW)
    pltpu.sync_copy(ian_hbm.at[pl.ds(base, _BPW)], idxn_v.at[head])
    pltpu.sync_copy(iab_hbm.at[pl.ds(base, _BPW)], idxa_v.at[head])
    pltpu.sync_copy(mn_hbm.at[pl.ds(base, _BPW)], mn_v.at[head])
    pltpu.sync_copy(ma_hbm.at[pl.ds(base, _BPW)], ma_v.at[head])
    pltpu.sync_copy(qn_hbm.at[pl.ds(base, _BPW)], qn_v.at[head])
    pltpu.sync_copy(nn_hbm, nn_v)
    pltpu.sync_copy(na_hbm, na_v)
    for c in range(_BPWP // 16):
        sl = pl.ds(c * 16, 16)
        ixn = idxn_v[sl] & (_K - 1)
        ixa = idxa_v[sl] & (_K - 1)
        nsn = plsc.load_gather(nn_v, [ixn])
        nsa = plsc.load_gather(na_v, [ixa])
        qc = qn_v[sl]
        on_v[sl] = (qc - 2.0 * mn_v[sl] + nsn) * (1.0 / _D)
        oa_v[sl] = (qc - 2.0 * ma_v[sl] + nsa) * (1.0 / _D)
    pltpu.sync_copy(on_v.at[head], outn_hbm.at[pl.ds(base, _BPW)])
    pltpu.sync_copy(oa_v.at[head], outa_hbm.at[pl.ds(base, _BPW)])


@functools.cache
def _sc_loss_kernel():
    mesh = plsc.VectorSubcoreMesh(core_axis_name="c", subcore_axis_name="s")
    return pl.kernel(
        _sc_loss_body,
        out_type=(jax.ShapeDtypeStruct((_B,), jnp.float32),
                  jax.ShapeDtypeStruct((_B,), jnp.float32)),
        mesh=mesh,
        scratch_types=[
            pltpu.VMEM((_BPWP,), jnp.int32),
            pltpu.VMEM((_BPWP,), jnp.int32),
            pltpu.VMEM((_BPWP,), jnp.float32),
            pltpu.VMEM((_BPWP,), jnp.float32),
            pltpu.VMEM((_BPWP,), jnp.float32),
            pltpu.VMEM((_K,), jnp.float32),
            pltpu.VMEM((_K,), jnp.float32),
            pltpu.VMEM((_BPWP,), jnp.float32),
            pltpu.VMEM((_BPWP,), jnp.float32),
        ],
        compiler_params=pltpu.CompilerParams(needs_layout_passes=False),
    )


def kernel(ref_nor, ref_abn, nor_keys, abn_keys, W_conv, b_conv, W_p, b_p,
           W_pm, b_pm, epoch, isTrain):
    wc = jnp.transpose(W_conv, (2, 1, 0))
    ones_col = jnp.ones((_K, 1), jnp.bfloat16)
    nkab = jnp.concatenate([nor_keys.astype(jnp.bfloat16), ones_col], axis=1)
    akab = jnp.concatenate([abn_keys.astype(jnp.bfloat16), ones_col], axis=1)
    coli = jnp.arange(_K, dtype=jnp.int32)
    colhl = jnp.stack([(coli >> 7).astype(jnp.bfloat16),
                       (coli & 127).astype(jnp.bfloat16)], axis=0)
    bc = b_conv.reshape(1, _D)
    wp = W_p.reshape(1, _D)
    bp = b_p.reshape(1, 1)
    wpm1 = W_pm[:, :_D].reshape(1, _D)
    wpm2 = W_pm[:, _D:].reshape(1, _D)
    bpm = b_pm.reshape(1, 1)

    rn, p, up, ia, ib, mn, ma, qn = _tc_call(
        ref_nor, nor_keys, abn_keys, nkab, akab,
        colhl, wc, bc, wp, bp, wpm1, wpm2, bpm)

    normn = jnp.sum(nor_keys * nor_keys, axis=1)
    norma = jnp.sum(abn_keys * abn_keys, axis=1)
    ln, la = _sc_loss_kernel()(mn.reshape(_B), ma.reshape(_B), qn.reshape(_B),
                               ia.reshape(_B), ib.reshape(_B), normn, norma)

    return (p.reshape(_BS, _N), up.reshape(_BS, _N), ln, la, rn)


# speedup vs baseline: 1.0367x; 1.0132x over previous
import functools

import jax
import jax.numpy as jnp
from jax import lax
from jax.experimental import pallas as pl
from jax.experimental.pallas import tpu as pltpu, tpu_sc as plsc

_BS = 32
_N = 200
_D = 128
_F = 512
_K = 2048
_B = _BS * _N
_NW = 32
_BPW = _B // _NW


def _tc_body(x_ref, nk_ref, ak_ref, nkab_ref, akab_ref, colhl_ref,
             wc_ref, bc_ref, wp_ref, bp_ref,
             wpm1_ref, wpm2_ref, bpm_ref,
             rn_ref, p_ref, up_ref, ia_ref, ib_ref, mn_ref, ma_ref, qn_ref):
    x = x_ref[0]
    z0 = jnp.dot(x, wc_ref[0], preferred_element_type=jnp.float32)
    z1 = jnp.dot(x, wc_ref[1], preferred_element_type=jnp.float32)
    z2 = jnp.dot(x, wc_ref[2], preferred_element_type=jnp.float32)
    zrow = jnp.zeros((1, _D), jnp.float32)
    y = z1 + jnp.concatenate([zrow, z0[:-1]], axis=0) \
           + jnp.concatenate([z2[1:], zrow], axis=0)
    q = jnp.maximum(y + bc_ref[...], 0.0)
    rn_ref[0] = q

    nk = nk_ref[...]
    ak = ak_ref[...]
    dn = (((1,), (1,)), ((), ()))
    sn = lax.dot_general(q, nk, dn, preferred_element_type=jnp.float32)
    sa = lax.dot_general(q, ak, dn, preferred_element_type=jnp.float32)

    mn = jnp.max(sn, axis=1, keepdims=True)
    ma = jnp.max(sa, axis=1, keepdims=True)
    maskn = jnp.where(sn >= mn, 1.0, 0.0).astype(jnp.bfloat16)
    maska = jnp.where(sa >= ma, 1.0, 0.0).astype(jnp.bfloat16)
    colhl = colhl_ref[...]
    rn_idx = lax.dot_general(maskn, colhl, dn, preferred_element_type=jnp.float32)
    ra_idx = lax.dot_general(maska, colhl, dn, preferred_element_type=jnp.float32)
    iaf = 128.0 * rn_idx[:, 0:1] + rn_idx[:, 1:2]
    ibf = 128.0 * ra_idx[:, 0:1] + ra_idx[:, 1:2]
    ia = jnp.clip(iaf, 0.0, float(_K - 1)).astype(jnp.int32)
    ib = jnp.clip(ibf, 0.0, float(_K - 1)).astype(jnp.int32)

    m = jnp.maximum(mn, ma)
    pnb = jnp.exp(sn - m).astype(jnp.bfloat16)
    pab = jnp.exp(sa - m).astype(jnp.bfloat16)
    cmn = jnp.dot(pnb, nkab_ref[...], preferred_element_type=jnp.float32)
    cma = jnp.dot(pab, akab_ref[...], preferred_element_type=jnp.float32)
    l = cmn[:, _D:] + cma[:, _D:]
    cm = (cmn[:, :_D] + cma[:, :_D]) / l

    sp = jnp.sum(q * wp_ref[...], axis=1, keepdims=True) + bp_ref[...]
    p = 1.0 / (1.0 + jnp.exp(-sp))
    su = (jnp.sum(q * wpm1_ref[...], axis=1, keepdims=True)
          + jnp.sum(cm * wpm2_ref[...], axis=1, keepdims=True) + bpm_ref[...])
    up = 1.0 / (1.0 + jnp.exp(-su))

    p_ref[0] = p
    up_ref[0] = up
    ia_ref[0] = ia
    ib_ref[0] = ib
    mn_ref[0] = mn
    ma_ref[0] = ma
    qn_ref[0] = jnp.sum(q * q, axis=1, keepdims=True)


def _tc_call(x, nk, ak, nkab, akab, colhl, wc, bc, wp, bp, wpm1, wpm2, bpm):
    const2 = lambda b: (0, 0)
    const3 = lambda b: (0, 0, 0)
    return pl.pallas_call(
        _tc_body,
        grid=(_BS,),
        in_specs=[
            pl.BlockSpec((1, _N, _F), lambda b: (b, 0, 0)),
            pl.BlockSpec((_K, _D), const2),
            pl.BlockSpec((_K, _D), const2),
            pl.BlockSpec((_K, _D + 1), const2),
            pl.BlockSpec((_K, _D + 1), const2),
            pl.BlockSpec((2, _K), const2),
            pl.BlockSpec((3, _F, _D), const3),
            pl.BlockSpec((1, _D), const2),
            pl.BlockSpec((1, _D), const2),
            pl.BlockSpec((1, 1), const2),
            pl.BlockSpec((1, _D), const2),
            pl.BlockSpec((1, _D), const2),
            pl.BlockSpec((1, 1), const2),
        ],
        out_specs=[
            pl.BlockSpec((1, _N, _D), lambda b: (b, 0, 0)),
            pl.BlockSpec((1, _N, 1), lambda b: (b, 0, 0)),
            pl.BlockSpec((1, _N, 1), lambda b: (b, 0, 0)),
            pl.BlockSpec((1, _N, 1), lambda b: (b, 0, 0)),
            pl.BlockSpec((1, _N, 1), lambda b: (b, 0, 0)),
            pl.BlockSpec((1, _N, 1), lambda b: (b, 0, 0)),
            pl.BlockSpec((1, _N, 1), lambda b: (b, 0, 0)),
            pl.BlockSpec((1, _N, 1), lambda b: (b, 0, 0)),
        ],
        out_shape=[
            jax.ShapeDtypeStruct((_BS, _N, _D), jnp.float32),
            jax.ShapeDtypeStruct((_BS, _N, 1), jnp.float32),
            jax.ShapeDtypeStruct((_BS, _N, 1), jnp.float32),
            jax.ShapeDtypeStruct((_BS, _N, 1), jnp.int32),
            jax.ShapeDtypeStruct((_BS, _N, 1), jnp.int32),
            jax.ShapeDtypeStruct((_BS, _N, 1), jnp.float32),
            jax.ShapeDtypeStruct((_BS, _N, 1), jnp.float32),
            jax.ShapeDtypeStruct((_BS, _N, 1), jnp.float32),
        ],
        compiler_params=pltpu.CompilerParams(
            dimension_semantics=("arbitrary",),
        ),
    )(x, nk, ak, nkab, akab, colhl, wc, bc, wp, bp, wpm1, wpm2, bpm)


_BPWP = 208


def _sc_loss_body(mn_hbm, ma_hbm, qn_hbm, ian_hbm, iab_hbm, nn_hbm, na_hbm,
                  outn_hbm, outa_hbm,
                  idxn_v, idxa_v, mn_v, ma_v, qn_v, nn_v, na_v, on_v, oa_v,
                  sem):
    wid = lax.axis_index("s") * 2 + lax.axis_index("c")
    base = wid * _BPW
    head = pl.ds(0, _BPW)
    descs = [
        pltpu.async_copy(ian_hbm.at[pl.ds(base, _BPW)], idxn_v.at[head], sem),
        pltpu.async_copy(iab_hbm.at[pl.ds(base, _BPW)], idxa_v.at[head], sem),
        pltpu.async_copy(mn_hbm.at[pl.ds(base, _BPW)], mn_v.at[head], sem),
        pltpu.async_copy(ma_hbm.at[pl.ds(base, _BPW)], ma_v.at[head], sem),
        pltpu.async_copy(qn_hbm.at[pl.ds(base, _BPW)], qn_v.at[head], sem),
        pltpu.async_copy(nn_hbm, nn_v, sem),
        pltpu.async_copy(na_hbm, na_v, sem),
    ]
    for d in descs:
        d.wait()
    for c in range(_BPWP // 16):
        sl = pl.ds(c * 16, 16)
        ixn = idxn_v[sl] & (_K - 1)
        ixa = idxa_v[sl] & (_K - 1)
        nsn = plsc.load_gather(nn_v, [ixn])
        nsa = plsc.load_gather(na_v, [ixa])
        qc = qn_v[sl]
        on_v[sl] = (qc - 2.0 * mn_v[sl] + nsn) * (1.0 / _D)
        oa_v[sl] = (qc - 2.0 * ma_v[sl] + nsa) * (1.0 / _D)
    outs = [
        pltpu.async_copy(on_v.at[head], outn_hbm.at[pl.ds(base, _BPW)], sem),
        pltpu.async_copy(oa_v.at[head], outa_hbm.at[pl.ds(base, _BPW)], sem),
    ]
    for d in outs:
        d.wait()


@functools.cache
def _sc_loss_kernel():
    mesh = plsc.VectorSubcoreMesh(core_axis_name="c", subcore_axis_name="s")
    return pl.kernel(
        _sc_loss_body,
        out_type=(jax.ShapeDtypeStruct((_B,), jnp.float32),
                  jax.ShapeDtypeStruct((_B,), jnp.float32)),
        mesh=mesh,
        scratch_types=[
            pltpu.VMEM((_BPWP,), jnp.int32),
            pltpu.VMEM((_BPWP,), jnp.int32),
            pltpu.VMEM((_BPWP,), jnp.float32),
            pltpu.VMEM((_BPWP,), jnp.float32),
            pltpu.VMEM((_BPWP,), jnp.float32),
            pltpu.VMEM((_K,), jnp.float32),
            pltpu.VMEM((_K,), jnp.float32),
            pltpu.VMEM((_BPWP,), jnp.float32),
            pltpu.VMEM((_BPWP,), jnp.float32),
            pltpu.SemaphoreType.DMA,
        ],
        compiler_params=pltpu.CompilerParams(needs_layout_passes=False),
    )


def kernel(ref_nor, ref_abn, nor_keys, abn_keys, W_conv, b_conv, W_p, b_p,
           W_pm, b_pm, epoch, isTrain):
    wc = jnp.transpose(W_conv, (2, 1, 0))
    ones_col = jnp.ones((_K, 1), jnp.bfloat16)
    nkab = jnp.concatenate([nor_keys.astype(jnp.bfloat16), ones_col], axis=1)
    akab = jnp.concatenate([abn_keys.astype(jnp.bfloat16), ones_col], axis=1)
    coli = jnp.arange(_K, dtype=jnp.int32)
    colhl = jnp.stack([(coli >> 7).astype(jnp.bfloat16),
                       (coli & 127).astype(jnp.bfloat16)], axis=0)
    bc = b_conv.reshape(1, _D)
    wp = W_p.reshape(1, _D)
    bp = b_p.reshape(1, 1)
    wpm1 = W_pm[:, :_D].reshape(1, _D)
    wpm2 = W_pm[:, _D:].reshape(1, _D)
    bpm = b_pm.reshape(1, 1)

    rn, p, up, ia, ib, mn, ma, qn = _tc_call(
        ref_nor, nor_keys, abn_keys, nkab, akab,
        colhl, wc, bc, wp, bp, wpm1, wpm2, bpm)

    normn = jnp.sum(nor_keys * nor_keys, axis=1)
    norma = jnp.sum(abn_keys * abn_keys, axis=1)
    ln, la = _sc_loss_kernel()(mn.reshape(_B), ma.reshape(_B), qn.reshape(_B),
                               ia.reshape(_B), ib.reshape(_B), normn, norma)

    return (p.reshape(_BS, _N), up.reshape(_BS, _N), ln, la, rn)


# speedup vs baseline: 1.0378x; 1.0010x over previous
import functools

import jax
import jax.numpy as jnp
from jax import lax
from jax.experimental import pallas as pl
from jax.experimental.pallas import tpu as pltpu, tpu_sc as plsc

_BS = 32
_N = 200
_D = 128
_F = 512
_K = 2048
_B = _BS * _N
_NW = 32
_BPW = _B // _NW


def _tc_body(x_ref, nk_ref, ak_ref, nkab_ref, akab_ref, colhl_ref,
             wc_ref, bc_ref, wp_ref, bp_ref,
             wpm1_ref, wpm2_ref, bpm_ref,
             rn_ref, p_ref, up_ref, ia_ref, ib_ref, mn_ref, ma_ref, qn_ref):
    x = x_ref[0]
    z0 = jnp.dot(x, wc_ref[0], preferred_element_type=jnp.float32)
    z1 = jnp.dot(x, wc_ref[1], preferred_element_type=jnp.float32)
    z2 = jnp.dot(x, wc_ref[2], preferred_element_type=jnp.float32)
    zrow = jnp.zeros((1, _D), jnp.float32)
    y = z1 + jnp.concatenate([zrow, z0[:-1]], axis=0) \
           + jnp.concatenate([z2[1:], zrow], axis=0)
    q = jnp.maximum(y + bc_ref[...], 0.0)
    rn_ref[0] = q

    nk = nk_ref[...]
    ak = ak_ref[...]
    dn = (((1,), (1,)), ((), ()))
    sn = lax.dot_general(q, nk, dn, preferred_element_type=jnp.float32)
    sa = lax.dot_general(q, ak, dn, preferred_element_type=jnp.float32)

    mn = jnp.max(sn, axis=1, keepdims=True)
    ma = jnp.max(sa, axis=1, keepdims=True)
    maskn = jnp.where(sn >= mn, 1.0, 0.0).astype(jnp.bfloat16)
    maska = jnp.where(sa >= ma, 1.0, 0.0).astype(jnp.bfloat16)
    colhl = colhl_ref[...]
    rn_idx = lax.dot_general(maskn, colhl, dn, preferred_element_type=jnp.float32)
    ra_idx = lax.dot_general(maska, colhl, dn, preferred_element_type=jnp.float32)
    iaf = 128.0 * rn_idx[:, 0:1] + rn_idx[:, 1:2]
    ibf = 128.0 * ra_idx[:, 0:1] + ra_idx[:, 1:2]
    ia = jnp.clip(iaf, 0.0, float(_K - 1)).astype(jnp.int32)
    ib = jnp.clip(ibf, 0.0, float(_K - 1)).astype(jnp.int32)

    m = jnp.maximum(mn, ma)
    pnb = jnp.exp(sn - m).astype(jnp.bfloat16)
    pab = jnp.exp(sa - m).astype(jnp.bfloat16)
    cmn = jnp.dot(pnb, nkab_ref[...], preferred_element_type=jnp.float32)
    cma = jnp.dot(pab, akab_ref[...], preferred_element_type=jnp.float32)
    l = cmn[:, _D:] + cma[:, _D:]
    cm = (cmn[:, :_D] + cma[:, :_D]) / l

    sp = jnp.sum(q * wp_ref[...], axis=1, keepdims=True) + bp_ref[...]
    p = 1.0 / (1.0 + jnp.exp(-sp))
    su = (jnp.sum(q * wpm1_ref[...], axis=1, keepdims=True)
          + jnp.sum(cm * wpm2_ref[...], axis=1, keepdims=True) + bpm_ref[...])
    up = 1.0 / (1.0 + jnp.exp(-su))

    p_ref[0] = p
    up_ref[0] = up
    ia_ref[0] = ia
    ib_ref[0] = ib
    mn_ref[0] = mn
    ma_ref[0] = ma
    qn_ref[0] = jnp.sum(q * q, axis=1, keepdims=True)


def _tc_call(x, nk, ak, nkab, akab, colhl, wc, bc, wp, bp, wpm1, wpm2, bpm):
    const2 = lambda b: (0, 0)
    const3 = lambda b: (0, 0, 0)
    return pl.pallas_call(
        _tc_body,
        grid=(_BS,),
        in_specs=[
            pl.BlockSpec((1, _N, _F), lambda b: (b, 0, 0)),
            pl.BlockSpec((_K, _D), const2),
            pl.BlockSpec((_K, _D), const2),
            pl.BlockSpec((_K, _D + 1), const2),
            pl.BlockSpec((_K, _D + 1), const2),
            pl.BlockSpec((2, _K), const2),
            pl.BlockSpec((3, _F, _D), const3),
            pl.BlockSpec((1, _D), const2),
            pl.BlockSpec((1, _D), const2),
            pl.BlockSpec((1, 1), const2),
            pl.BlockSpec((1, _D), const2),
            pl.BlockSpec((1, _D), const2),
            pl.BlockSpec((1, 1), const2),
        ],
        out_specs=[
            pl.BlockSpec((1, _N, _D), lambda b: (b, 0, 0)),
            pl.BlockSpec((1, _N, 1), lambda b: (b, 0, 0)),
            pl.BlockSpec((1, _N, 1), lambda b: (b, 0, 0)),
            pl.BlockSpec((1, _N, 1), lambda b: (b, 0, 0)),
            pl.BlockSpec((1, _N, 1), lambda b: (b, 0, 0)),
            pl.BlockSpec((1, _N, 1), lambda b: (b, 0, 0)),
            pl.BlockSpec((1, _N, 1), lambda b: (b, 0, 0)),
            pl.BlockSpec((1, _N, 1), lambda b: (b, 0, 0)),
        ],
        out_shape=[
            jax.ShapeDtypeStruct((_BS, _N, _D), jnp.float32),
            jax.ShapeDtypeStruct((_BS, _N, 1), jnp.float32),
            jax.ShapeDtypeStruct((_BS, _N, 1), jnp.float32),
            jax.ShapeDtypeStruct((_BS, _N, 1), jnp.int32),
            jax.ShapeDtypeStruct((_BS, _N, 1), jnp.int32),
            jax.ShapeDtypeStruct((_BS, _N, 1), jnp.float32),
            jax.ShapeDtypeStruct((_BS, _N, 1), jnp.float32),
            jax.ShapeDtypeStruct((_BS, _N, 1), jnp.float32),
        ],
        compiler_params=pltpu.CompilerParams(
            dimension_semantics=("parallel",),
            vmem_limit_bytes=100 * 1024 * 1024,
        ),
    )(x, nk, ak, nkab, akab, colhl, wc, bc, wp, bp, wpm1, wpm2, bpm)


_BPWP = 208


def _sc_loss_body(mn_hbm, ma_hbm, qn_hbm, ian_hbm, iab_hbm, nn_hbm, na_hbm,
                  outn_hbm, outa_hbm,
                  idxn_v, idxa_v, mn_v, ma_v, qn_v, nn_v, na_v, on_v, oa_v,
                  sem):
    wid = lax.axis_index("s") * 2 + lax.axis_index("c")
    base = wid * _BPW
    head = pl.ds(0, _BPW)
    descs = [
        pltpu.async_copy(ian_hbm.at[pl.ds(base, _BPW)], idxn_v.at[head], sem),
        pltpu.async_copy(iab_hbm.at[pl.ds(base, _BPW)], idxa_v.at[head], sem),
        pltpu.async_copy(mn_hbm.at[pl.ds(base, _BPW)], mn_v.at[head], sem),
        pltpu.async_copy(ma_hbm.at[pl.ds(base, _BPW)], ma_v.at[head], sem),
        pltpu.async_copy(qn_hbm.at[pl.ds(base, _BPW)], qn_v.at[head], sem),
        pltpu.async_copy(nn_hbm, nn_v, sem),
        pltpu.async_copy(na_hbm, na_v, sem),
    ]
    for d in descs:
        d.wait()
    for c in range(_BPWP // 16):
        sl = pl.ds(c * 16, 16)
        ixn = idxn_v[sl] & (_K - 1)
        ixa = idxa_v[sl] & (_K - 1)
        nsn = plsc.load_gather(nn_v, [ixn])
        nsa = plsc.load_gather(na_v, [ixa])
        qc = qn_v[sl]
        on_v[sl] = (qc - 2.0 * mn_v[sl] + nsn) * (1.0 / _D)
        oa_v[sl] = (qc - 2.0 * ma_v[sl] + nsa) * (1.0 / _D)
    outs = [
        pltpu.async_copy(on_v.at[head], outn_hbm.at[pl.ds(base, _BPW)], sem),
        pltpu.async_copy(oa_v.at[head], outa_hbm.at[pl.ds(base, _BPW)], sem),
    ]
    for d in outs:
        d.wait()


@functools.cache
def _sc_loss_kernel():
    mesh = plsc.VectorSubcoreMesh(core_axis_name="c", subcore_axis_name="s")
    return pl.kernel(
        _sc_loss_body,
        out_type=(jax.ShapeDtypeStruct((_B,), jnp.float32),
                  jax.ShapeDtypeStruct((_B,), jnp.float32)),
        mesh=mesh,
        scratch_types=[
            pltpu.VMEM((_BPWP,), jnp.int32),
            pltpu.VMEM((_BPWP,), jnp.int32),
            pltpu.VMEM((_BPWP,), jnp.float32),
            pltpu.VMEM((_BPWP,), jnp.float32),
            pltpu.VMEM((_BPWP,), jnp.float32),
            pltpu.VMEM((_K,), jnp.float32),
            pltpu.VMEM((_K,), jnp.float32),
            pltpu.VMEM((_BPWP,), jnp.float32),
            pltpu.VMEM((_BPWP,), jnp.float32),
            pltpu.SemaphoreType.DMA,
        ],
        compiler_params=pltpu.CompilerParams(needs_layout_passes=False),
    )


def kernel(ref_nor, ref_abn, nor_keys, abn_keys, W_conv, b_conv, W_p, b_p,
           W_pm, b_pm, epoch, isTrain):
    wc = jnp.transpose(W_conv, (2, 1, 0))
    ones_col = jnp.ones((_K, 1), jnp.bfloat16)
    nkab = jnp.concatenate([nor_keys.astype(jnp.bfloat16), ones_col], axis=1)
    akab = jnp.concatenate([abn_keys.astype(jnp.bfloat16), ones_col], axis=1)
    coli = jnp.arange(_K, dtype=jnp.int32)
    colhl = jnp.stack([(coli >> 7).astype(jnp.bfloat16),
                       (coli & 127).astype(jnp.bfloat16)], axis=0)
    bc = b_conv.reshape(1, _D)
    wp = W_p.reshape(1, _D)
    bp = b_p.reshape(1, 1)
    wpm1 = W_pm[:, :_D].reshape(1, _D)
    wpm2 = W_pm[:, _D:].reshape(1, _D)
    bpm = b_pm.reshape(1, 1)

    rn, p, up, ia, ib, mn, ma, qn = _tc_call(
        ref_nor, nor_keys, abn_keys, nkab, akab,
        colhl, wc, bc, wp, bp, wpm1, wpm2, bpm)

    normn = jnp.sum(nor_keys * nor_keys, axis=1)
    norma = jnp.sum(abn_keys * abn_keys, axis=1)
    ln, la = _sc_loss_kernel()(mn.reshape(_B), ma.reshape(_B), qn.reshape(_B),
                               ia.reshape(_B), ib.reshape(_B), normn, norma)

    return (p.reshape(_BS, _N), up.reshape(_BS, _N), ln, la, rn)


# speedup vs baseline: 1.1508x; 1.1088x over previous
import functools

import jax
import jax.numpy as jnp
from jax import lax
from jax.experimental import pallas as pl
from jax.experimental.pallas import tpu as pltpu, tpu_sc as plsc

_BS = 32
_N = 200
_D = 128
_F = 512
_K = 2048
_B = _BS * _N
_NW = 32
_BPW = _B // _NW


def _tc_body(x_ref, nk_ref, ak_ref, nkab_ref, akab_ref, colhl_ref,
             wc_ref, bc_ref, wp_ref, bp_ref,
             wpm1_ref, wpm2_ref, bpm_ref,
             rn_ref, st_ref):
    x = x_ref[0]
    z0 = jnp.dot(x, wc_ref[0], preferred_element_type=jnp.float32)
    z1 = jnp.dot(x, wc_ref[1], preferred_element_type=jnp.float32)
    z2 = jnp.dot(x, wc_ref[2], preferred_element_type=jnp.float32)
    zrow = jnp.zeros((1, _D), jnp.float32)
    y = z1 + jnp.concatenate([zrow, z0[:-1]], axis=0) \
           + jnp.concatenate([z2[1:], zrow], axis=0)
    q = jnp.maximum(y + bc_ref[...], 0.0)
    rn_ref[0] = q

    nk = nk_ref[...]
    ak = ak_ref[...]
    dn = (((1,), (1,)), ((), ()))
    sn = lax.dot_general(q, nk, dn, preferred_element_type=jnp.float32)
    sa = lax.dot_general(q, ak, dn, preferred_element_type=jnp.float32)

    mn = jnp.max(sn, axis=1, keepdims=True)
    ma = jnp.max(sa, axis=1, keepdims=True)
    maskn = jnp.where(sn >= mn, 1.0, 0.0).astype(jnp.bfloat16)
    maska = jnp.where(sa >= ma, 1.0, 0.0).astype(jnp.bfloat16)
    colhl = colhl_ref[...]
    rn_idx = lax.dot_general(maskn, colhl, dn, preferred_element_type=jnp.float32)
    ra_idx = lax.dot_general(maska, colhl, dn, preferred_element_type=jnp.float32)
    iaf = 128.0 * rn_idx[:, 0:1] + rn_idx[:, 1:2]
    ibf = 128.0 * ra_idx[:, 0:1] + ra_idx[:, 1:2]
    ia = jnp.clip(iaf, 0.0, float(_K - 1)).astype(jnp.int32)
    ib = jnp.clip(ibf, 0.0, float(_K - 1)).astype(jnp.int32)

    m = jnp.maximum(mn, ma)
    pnb = jnp.exp(sn - m).astype(jnp.bfloat16)
    pab = jnp.exp(sa - m).astype(jnp.bfloat16)
    cmn = jnp.dot(pnb, nkab_ref[...], preferred_element_type=jnp.float32)
    cma = jnp.dot(pab, akab_ref[...], preferred_element_type=jnp.float32)
    l = cmn[:, _D:] + cma[:, _D:]
    cm = (cmn[:, :_D] + cma[:, :_D]) / l

    sp = jnp.sum(q * wp_ref[...], axis=1, keepdims=True) + bp_ref[...]
    p = 1.0 / (1.0 + jnp.exp(-sp))
    su = (jnp.sum(q * wpm1_ref[...], axis=1, keepdims=True)
          + jnp.sum(cm * wpm2_ref[...], axis=1, keepdims=True) + bpm_ref[...])
    up = 1.0 / (1.0 + jnp.exp(-su))

    qn2 = jnp.sum(q * q, axis=1, keepdims=True)
    iab = lax.bitcast_convert_type(ia, jnp.float32)
    ibb = lax.bitcast_convert_type(ib, jnp.float32)
    zcol = jnp.zeros((_N, 1), jnp.float32)
    st_ref[0] = jnp.concatenate([p, up, mn, ma, qn2, iab, ibb, zcol], axis=1)


def _tc_call(x, nk, ak, nkab, akab, colhl, wc, bc, wp, bp, wpm1, wpm2, bpm):
    const2 = lambda b: (0, 0)
    const3 = lambda b: (0, 0, 0)
    return pl.pallas_call(
        _tc_body,
        grid=(_BS,),
        in_specs=[
            pl.BlockSpec((1, _N, _F), lambda b: (b, 0, 0)),
            pl.BlockSpec((_K, _D), const2),
            pl.BlockSpec((_K, _D), const2),
            pl.BlockSpec((_K, _D + 1), const2),
            pl.BlockSpec((_K, _D + 1), const2),
            pl.BlockSpec((2, _K), const2),
            pl.BlockSpec((3, _F, _D), const3),
            pl.BlockSpec((1, _D), const2),
            pl.BlockSpec((1, _D), const2),
            pl.BlockSpec((1, 1), const2),
            pl.BlockSpec((1, _D), const2),
            pl.BlockSpec((1, _D), const2),
            pl.BlockSpec((1, 1), const2),
        ],
        out_specs=[
            pl.BlockSpec((1, _N, _D), lambda b: (b, 0, 0)),
            pl.BlockSpec((1, _N, 8), lambda b: (b, 0, 0)),
        ],
        out_shape=[
            jax.ShapeDtypeStruct((_BS, _N, _D), jnp.float32),
            jax.ShapeDtypeStruct((_BS, _N, 8), jnp.float32),
        ],
        compiler_params=pltpu.CompilerParams(
            dimension_semantics=("parallel",),
            vmem_limit_bytes=100 * 1024 * 1024,
        ),
    )(x, nk, ak, nkab, akab, colhl, wc, bc, wp, bp, wpm1, wpm2, bpm)


_BPWP = 208


_BPWP = 208


def _sc_loss_body(st_hbm, nn_hbm, na_hbm, outn_hbm, outa_hbm,
                  st_v, nn_v, na_v, on_v, oa_v, sem):
    wid = lax.axis_index("s") * 2 + lax.axis_index("c")
    base = wid * _BPW
    head = pl.ds(0, _BPW)
    descs = [
        pltpu.async_copy(st_hbm.at[pl.ds(base * 8, _BPW * 8)],
                         st_v.at[pl.ds(0, _BPW * 8)], sem),
        pltpu.async_copy(nn_hbm, nn_v, sem),
        pltpu.async_copy(na_hbm, na_v, sem),
    ]
    for d in descs:
        d.wait()
    lane8 = lax.broadcasted_iota(jnp.int32, (16,), 0) * 8
    for c in range(_BPWP // 16):
        sl = pl.ds(c * 16, 16)
        fb = c * 128
        mnv = plsc.load_gather(st_v, [lane8 + (fb + 2)])
        mav = plsc.load_gather(st_v, [lane8 + (fb + 3)])
        qnv = plsc.load_gather(st_v, [lane8 + (fb + 4)])
        iav = plsc.bitcast(plsc.load_gather(st_v, [lane8 + (fb + 5)]),
                           jnp.int32) & (_K - 1)
        ibv = plsc.bitcast(plsc.load_gather(st_v, [lane8 + (fb + 6)]),
                           jnp.int32) & (_K - 1)
        nsn = plsc.load_gather(nn_v, [iav])
        nsa = plsc.load_gather(na_v, [ibv])
        on_v[sl] = (qnv - 2.0 * mnv + nsn) * (1.0 / _D)
        oa_v[sl] = (qnv - 2.0 * mav + nsa) * (1.0 / _D)
    outs = [
        pltpu.async_copy(on_v.at[head], outn_hbm.at[pl.ds(base, _BPW)], sem),
        pltpu.async_copy(oa_v.at[head], outa_hbm.at[pl.ds(base, _BPW)], sem),
    ]
    for d in outs:
        d.wait()


@functools.cache
def _sc_loss_kernel():
    mesh = plsc.VectorSubcoreMesh(core_axis_name="c", subcore_axis_name="s")
    return pl.kernel(
        _sc_loss_body,
        out_type=(jax.ShapeDtypeStruct((_B,), jnp.float32),
                  jax.ShapeDtypeStruct((_B,), jnp.float32)),
        mesh=mesh,
        scratch_types=[
            pltpu.VMEM((_BPWP * 8,), jnp.float32),
            pltpu.VMEM((_K,), jnp.float32),
            pltpu.VMEM((_K,), jnp.float32),
            pltpu.VMEM((_BPWP,), jnp.float32),
            pltpu.VMEM((_BPWP,), jnp.float32),
            pltpu.SemaphoreType.DMA,
        ],
        compiler_params=pltpu.CompilerParams(needs_layout_passes=False),
    )


def kernel(ref_nor, ref_abn, nor_keys, abn_keys, W_conv, b_conv, W_p, b_p,
           W_pm, b_pm, epoch, isTrain):
    wc = jnp.transpose(W_conv, (2, 1, 0))
    ones_col = jnp.ones((_K, 1), jnp.bfloat16)
    nkab = jnp.concatenate([nor_keys.astype(jnp.bfloat16), ones_col], axis=1)
    akab = jnp.concatenate([abn_keys.astype(jnp.bfloat16), ones_col], axis=1)
    coli = jnp.arange(_K, dtype=jnp.int32)
    colhl = jnp.stack([(coli >> 7).astype(jnp.bfloat16),
                       (coli & 127).astype(jnp.bfloat16)], axis=0)
    bc = b_conv.reshape(1, _D)
    wp = W_p.reshape(1, _D)
    bp = b_p.reshape(1, 1)
    wpm1 = W_pm[:, :_D].reshape(1, _D)
    wpm2 = W_pm[:, _D:].reshape(1, _D)
    bpm = b_pm.reshape(1, 1)

    rn, st8 = _tc_call(
        ref_nor, nor_keys, abn_keys, nkab, akab,
        colhl, wc, bc, wp, bp, wpm1, wpm2, bpm)

    normn = jnp.sum(nor_keys * nor_keys, axis=1)
    norma = jnp.sum(abn_keys * abn_keys, axis=1)
    ln, la = _sc_loss_kernel()(st8.reshape(_B * 8), normn, norma)

    return (st8[:, :, 0], st8[:, :, 1], ln, la, rn)
